# baseline (speedup 1.0000x reference)
import functools

import jax
import jax.numpy as jnp
from jax import lax
from jax.experimental import pallas as pl
from jax.experimental.pallas import tpu as pltpu
from jax.experimental.pallas import tpu_sc as plsc

B, HW, D, T, NCAT = 4, 4096, 128, 64, 21
THRESH = 250.0

NW = 32
SC_PX = 2048
PXW = SC_PX // NW
PXC = 8
NPXC = PXW // PXC
TCH = 32
L = 16
F32 = jnp.float32
I32 = jnp.int32

TROW = PXC * D
TSTR = TROW + 8
TB = TCH * TSTR
HALF = L * TSTR
FB = B * PXC * D
DSTR = T + 8
FLAT = B * HW
PSTR = 2 * PXC * NCAT + 8
PLEN = 2 * PXC * NCAT


def _c(v):
    return jnp.full((L,), v, I32)


def _body(frame, tpl, clsa, pred_o, maski_o, ncls_o, mind_o, ucls_o,
          tbuf, fbuf, distbuf, clsv, predb, minb, maskb, nclsb, uclsb,
          tsem, fsem, osem):
    wid = lax.axis_index("s") * 2 + lax.axis_index("c")
    pxbase = wid * PXW
    iota = lax.iota(I32, L)
    tpat = iota * TSTR
    hi8 = lax.shift_right_logical(iota, 3)
    lo8 = jnp.bitwise_and(iota, 7)
    dpat = hi8 * (PXC * DSTR) + lo8 * DSTR
    spat = hi8 * (2 * PXC) + lo8
    ppat = hi8 * PSTR + lo8 * NCAT

    pltpu.sync_copy(clsa, clsv)

    def fire_tpl(pxc, tgc, tp):
        px0 = pxbase + pxc * PXC

        def row(i, _):
            pltpu.async_copy(
                tpl.at[pl.ds((tgc * TCH + i) * (HW * D) + px0 * D, TROW)],
                tbuf.at[pl.ds(tp * TB + i * TSTR, TROW)], tsem.at[tp])
            return 0

        lax.fori_loop(0, TCH, row, 0, unroll=4)

    def wait_tpl(tp):
        pltpu.make_async_copy(
            tpl.at[pl.ds(0, TCH * TROW)],
            tbuf.at[pl.ds(tp * TB, TCH * TROW)], tsem.at[tp]).wait()

    def fire_frame(pxc, fp):
        px0 = pxbase + pxc * PXC
        for b in range(B):
            pltpu.async_copy(
                frame.at[pl.ds(b * (HW * D) + px0 * D, PXC * D)],
                fbuf.at[pl.ds(fp * FB + b * PXC * D, PXC * D)], fsem.at[fp])

    def wait_frame(fp):
        pltpu.make_async_copy(frame.at[pl.ds(0, FB)],
                              fbuf.at[pl.ds(fp * FB, FB)], fsem.at[fp]).wait()

    def out_copies(op, px0):
        cps = []
        for b in range(B):
            cps.append(pltpu.make_async_copy(
                predb.at[pl.ds(op * B * PSTR + b * PSTR, PLEN)],
                pred_o.at[pl.ds((b * HW + px0) * NCAT, PLEN)],
                osem.at[op]))
            for buf, out in ((minb, mind_o), (maskb, maski_o),
                             (nclsb, ncls_o), (uclsb, ucls_o)):
                cps.append(pltpu.make_async_copy(
                    buf.at[pl.ds(op * B * 2 * PXC + b * 2 * PXC, 2 * PXC)],
                    out.at[pl.ds(b * HW + px0, 2 * PXC)], osem.at[op]))
        return cps

    def compute_chunk(tp, fp, tgc):
        def px_step(px, _):
            idx0 = tpat + _c(tp * TB + px * D)
            fb0 = fp * FB + px * D
            doff = px * DSTR + tgc * TCH

            def k_step(k, carry):
                idx, accs = carry
                fvecs = [fbuf[pl.ds(fb0 + b * (PXC * D) + k * 8, L)]
                         for b in range(B)]
                part = [None] * (2 * B)
                for j in range(8):
                    a0 = idx + _c(j)
                    a1 = a0 + _c(HALF)
                    tv0 = plsc.load_gather(tbuf, [a0])
                    tv1 = plsc.load_gather(tbuf, [a1])
                    for b in range(B):
                        fs = fvecs[b][j]
                        d0 = fs - tv0
                        d1 = fs - tv1
                        if j == 0:
                            part[2 * b] = d0 * d0
                            part[2 * b + 1] = d1 * d1
                        else:
                            part[2 * b] = part[2 * b] + d0 * d0
                            part[2 * b + 1] = part[2 * b + 1] + d1 * d1
                accs = tuple(a + p for a, p in zip(accs, part))
                return idx + _c(8), accs

            zero = jnp.zeros((L,), F32)
            _, accs = lax.fori_loop(0, D // 8, k_step,
                                    (idx0, (zero,) * (2 * B)))
            for b in range(B):
                for h in range(2):
                    distbuf[pl.ds(b * (PXC * DSTR) + doff + h * L, L)] = \
                        accs[2 * b + h]
            return 0

        lax.fori_loop(0, PXC, px_step, 0)

    def phase2(op, h):
        for b0 in (0, 2):
            base = dpat + _c(b0 * (PXC * DSTR))
            bd = jnp.full((L,), jnp.inf, F32)
            bi = jnp.zeros((L,), I32)

            def t_step(q, carry):
                bd, bi = carry
                t = 4 * q
                v0 = plsc.load_gather(distbuf, [base + t])
                v1 = plsc.load_gather(distbuf, [base + (t + 1)])
                v2 = plsc.load_gather(distbuf, [base + (t + 2)])
                v3 = plsc.load_gather(distbuf, [base + (t + 3)])
                i01 = jnp.where(v1 < v0, t + 1, t)
                m01 = jnp.minimum(v0, v1)
                i23 = jnp.where(v3 < v2, t + 3, t + 2)
                m23 = jnp.minimum(v2, v3)
                lt2 = m23 < m01
                m4 = jnp.where(lt2, m23, m01)
                i4 = jnp.where(lt2, i23, i01)
                lt = m4 < bd
                return jnp.where(lt, m4, bd), jnp.where(lt, i4, bi)

            bd, bi = lax.fori_loop(0, T // 4, t_step, (bd, bi))
            mask = bd <= THRESH
            cls = plsc.load_gather(clsv, [bi])
            so = spat + _c(op * B * 2 * PXC + b0 * 2 * PXC + h * PXC)
            plsc.store_scatter(minb, [so], bd)
            plsc.store_scatter(maskb, [so], jnp.where(mask, 1, 0).astype(I32))
            plsc.store_scatter(nclsb, [so],
                               jnp.where(mask, cls, NCAT - 1).astype(I32))
            plsc.store_scatter(uclsb, [so], cls)
            po = ppat + _c(op * B * PSTR + b0 * PSTR + h * PXC * NCAT)
            for c in range(NCAT):
                pv = jnp.where((cls == c) & mask, 1.0, 0.0).astype(F32)
                plsc.store_scatter(predb, [po + _c(c)], pv)

    fire_frame(0, 0)
    fire_tpl(0, 0, 0)

    def outer(i, _):
        for s in range(8):
            tp = s % 2
            tgc = s % 2
            q = (s // 2) % 2
            op = s // 4
            if tgc == 0:
                wait_frame(q)
                if s // 2 == 3:
                    @pl.when(i < NPXC // 4 - 1)
                    def _():
                        fire_frame(4 * i + 4, 1 - q)
                else:
                    fire_frame(4 * i + s // 2 + 1, 1 - q)
            wait_tpl(tp)
            if s == 7:
                @pl.when(i < NPXC // 4 - 1)
                def _():
                    fire_tpl(4 * i + 4, 0, 0)
            else:
                fire_tpl(4 * i + (s + 1) // 2, (s + 1) % 2, 1 - tp)
            compute_chunk(tp, q, tgc)
            if tgc == 1:
                if s % 4 == 1:
                    @pl.when(i >= 1)
                    def _():
                        for cp in out_copies(op, 0):
                            cp.wait()
                phase2(op, (s // 2) % 2)
                if s % 4 == 3:
                    px0 = pxbase + (2 * i + s // 4) * 2 * PXC
                    for cp in out_copies(op, px0):
                        cp.start()
        return 0

    lax.fori_loop(0, NPXC // 4, outer, 0)

    for op in range(2):
        for cp in out_copies(op, 0):
            cp.wait()


@jax.jit
def _nn_classify(frame, tpl, clsa):
    mesh = plsc.VectorSubcoreMesh(core_axis_name="c", subcore_axis_name="s")
    fn = functools.partial(
        pl.kernel,
        out_type=(
            jax.ShapeDtypeStruct((FLAT * NCAT,), F32),
            jax.ShapeDtypeStruct((FLAT,), I32),
            jax.ShapeDtypeStruct((FLAT,), I32),
            jax.ShapeDtypeStruct((FLAT,), F32),
            jax.ShapeDtypeStruct((FLAT,), I32),
        ),
        mesh=mesh,
        compiler_params=pltpu.CompilerParams(needs_layout_passes=False),
        scratch_types=[
            pltpu.VMEM((2 * TB,), F32),
            pltpu.VMEM((2 * FB + 8,), F32),
            pltpu.VMEM((B * PXC * DSTR,), F32),
            pltpu.VMEM((T,), I32),
            pltpu.VMEM((2 * B * PSTR,), F32),
            pltpu.VMEM((2 * B * 2 * PXC,), F32),
            pltpu.VMEM((2 * B * 2 * PXC,), I32),
            pltpu.VMEM((2 * B * 2 * PXC,), I32),
            pltpu.VMEM((2 * B * 2 * PXC,), I32),
            pltpu.SemaphoreType.DMA((2,)),
            pltpu.SemaphoreType.DMA((2,)),
            pltpu.SemaphoreType.DMA((2,)),
        ],
    )(_body)
    return fn(frame, tpl, clsa)


TC_PX = HW - SC_PX
TC_TILE = 512
TC_NT = TC_PX // TC_TILE
TC_OFF = SC_PX // TC_TILE


def _tc_body(cls_ref, frame_ref, tpl_ref,
             pred_o, maski_o, ncls_o, mind_o, ucls_o, best, bestc):
    t = pl.program_id(1)
    tm = lax.broadcast_in_dim(tpl_ref[...], (B, TC_TILE, D), (0, 1, 2))
    diff = frame_ref[...] - tm
    d2 = jnp.sum(diff * diff, axis=2)
    ct = jnp.full((B, TC_TILE), cls_ref[t], I32)

    @pl.when(t == 0)
    def _():
        best[...] = d2
        bestc[...] = ct

    @pl.when(t > 0)
    def _():
        upd = d2 < best[...]
        best[...] = jnp.where(upd, d2, best[...])
        bestc[...] = jnp.where(upd, ct, bestc[...])

    @pl.when(t == T - 1)
    def _():
        bd = best[...]
        cls = bestc[...]
        m = bd <= THRESH
        mind_o[...] = bd
        maski_o[...] = m.astype(I32)
        ncls_o[...] = jnp.where(m, cls, NCAT - 1)
        ucls_o[...] = cls
        clsb = lax.broadcast_in_dim(cls, (B, NCAT, TC_TILE), (0, 2))
        mb = lax.broadcast_in_dim(m, (B, NCAT, TC_TILE), (0, 2))
        ioc = lax.broadcasted_iota(I32, (B, NCAT, TC_TILE), 1)
        pred_o[...] = jnp.where(jnp.logical_and(clsb == ioc, mb),
                                1.0, 0.0).astype(F32)


@jax.jit
def _tc_classify(frame, tpl, clsa):
    return pl.pallas_call(
        _tc_body,
        grid=(TC_NT, T),
        in_specs=[
            pl.BlockSpec(memory_space=pltpu.SMEM),
            pl.BlockSpec((B, TC_TILE, D), lambda pt, t: (0, TC_OFF + pt, 0)),
            pl.BlockSpec((1, TC_TILE, D), lambda pt, t: (t, TC_OFF + pt, 0)),
        ],
        out_specs=[
            pl.BlockSpec((B, NCAT, TC_TILE), lambda pt, t: (0, 0, pt)),
            pl.BlockSpec((B, TC_TILE), lambda pt, t: (0, pt)),
            pl.BlockSpec((B, TC_TILE), lambda pt, t: (0, pt)),
            pl.BlockSpec((B, TC_TILE), lambda pt, t: (0, pt)),
            pl.BlockSpec((B, TC_TILE), lambda pt, t: (0, pt)),
        ],
        out_shape=(
            jax.ShapeDtypeStruct((B, NCAT, TC_PX), F32),
            jax.ShapeDtypeStruct((B, TC_PX), I32),
            jax.ShapeDtypeStruct((B, TC_PX), I32),
            jax.ShapeDtypeStruct((B, TC_PX), F32),
            jax.ShapeDtypeStruct((B, TC_PX), I32),
        ),
        scratch_shapes=[
            pltpu.VMEM((B, TC_TILE), F32),
            pltpu.VMEM((B, TC_TILE), I32),
        ],
    )(clsa, frame, tpl)


def kernel(frame_embeddings, templates, template_classes):
    pred, maski, ncls, mind, ucls = _nn_classify(
        frame_embeddings.reshape(B * HW * D),
        templates.reshape(T * HW * D),
        template_classes)
    tpred, tmaski, tncls, tmind, tucls = _tc_classify(
        frame_embeddings, templates, template_classes)
    cat = lambda a, b: jnp.concatenate([a, b], axis=1)
    return (
        cat(pred.reshape(B, HW, NCAT)[:, :SC_PX],
            jnp.transpose(tpred, (0, 2, 1))),
        cat(maski.reshape(B, HW)[:, :SC_PX], tmaski).astype(bool),
        cat(ncls.reshape(B, HW)[:, :SC_PX], tncls),
        cat(mind.reshape(B, HW)[:, :SC_PX], tmind),
        cat(ucls.reshape(B, HW)[:, :SC_PX], tucls),
    )


# speedup vs baseline: 1.7923x; 1.7923x over previous
import functools

import jax
import jax.numpy as jnp
from jax import lax
from jax.experimental import pallas as pl
from jax.experimental.pallas import tpu as pltpu
from jax.experimental.pallas import tpu_sc as plsc

B, HW, D, T, NCAT = 4, 4096, 128, 64, 21
THRESH = 250.0

NW = 32
SC_PX = 3072
PXW = SC_PX // NW
PXC = 8
NPXC = PXW // PXC
TCH = 32
L = 16
F32 = jnp.float32
I32 = jnp.int32

TROW = PXC * D
TSTR = TROW + 8
TB = TCH * TSTR
HALF = L * TSTR
FB = B * PXC * D
DSTR = T + 8
FLAT = B * HW
PSTR = 2 * PXC * NCAT + 8
PLEN = 2 * PXC * NCAT


def _c(v):
    return jnp.full((L,), v, I32)


def _body(frame, tpl, clsa, pred_o, maski_o, ncls_o, mind_o, ucls_o,
          tbuf, fbuf, distbuf, clsv, predb, minb, maskb, nclsb, uclsb,
          tsem, fsem, osem):
    wid = lax.axis_index("s") * 2 + lax.axis_index("c")
    pxbase = wid * PXW
    iota = lax.iota(I32, L)
    tpat = iota * TSTR
    hi8 = lax.shift_right_logical(iota, 3)
    lo8 = jnp.bitwise_and(iota, 7)
    dpat = hi8 * (PXC * DSTR) + lo8 * DSTR
    spat = hi8 * (2 * PXC) + lo8
    ppat = hi8 * PSTR + lo8 * NCAT

    pltpu.sync_copy(clsa, clsv)

    def fire_tpl(pxc, tgc, tp):
        px0 = pxbase + pxc * PXC

        def row(i, _):
            pltpu.async_copy(
                tpl.at[pl.ds((tgc * TCH + i) * (HW * D) + px0 * D, TROW)],
                tbuf.at[pl.ds(tp * TB + i * TSTR, TROW)], tsem.at[tp])
            return 0

        lax.fori_loop(0, TCH, row, 0, unroll=4)

    def wait_tpl(tp):
        pltpu.make_async_copy(
            tpl.at[pl.ds(0, TCH * TROW)],
            tbuf.at[pl.ds(tp * TB, TCH * TROW)], tsem.at[tp]).wait()

    def fire_frame(pxc, fp):
        px0 = pxbase + pxc * PXC
        for b in range(B):
            pltpu.async_copy(
                frame.at[pl.ds(b * (HW * D) + px0 * D, PXC * D)],
                fbuf.at[pl.ds(fp * FB + b * PXC * D, PXC * D)], fsem.at[fp])

    def wait_frame(fp):
        pltpu.make_async_copy(frame.at[pl.ds(0, FB)],
                              fbuf.at[pl.ds(fp * FB, FB)], fsem.at[fp]).wait()

    def out_copies(op, px0):
        cps = []
        for b in range(B):
            cps.append(pltpu.make_async_copy(
                predb.at[pl.ds(op * B * PSTR + b * PSTR, PLEN)],
                pred_o.at[pl.ds((b * HW + px0) * NCAT, PLEN)],
                osem.at[op]))
            for buf, out in ((minb, mind_o), (maskb, maski_o),
                             (nclsb, ncls_o), (uclsb, ucls_o)):
                cps.append(pltpu.make_async_copy(
                    buf.at[pl.ds(op * B * 2 * PXC + b * 2 * PXC, 2 * PXC)],
                    out.at[pl.ds(b * HW + px0, 2 * PXC)], osem.at[op]))
        return cps

    def compute_chunk(tp, fp, tgc):
        def px_step(px, _):
            idx0 = tpat + _c(tp * TB + px * D)
            fb0 = fp * FB + px * D
            doff = px * DSTR + tgc * TCH

            def k_step(k, carry):
                idx, accs = carry
                fvecs = [fbuf[pl.ds(fb0 + b * (PXC * D) + k * 8, L)]
                         for b in range(B)]
                part = [None] * (2 * B)
                for j in range(8):
                    a0 = idx + _c(j)
                    a1 = a0 + _c(HALF)
                    tv0 = plsc.load_gather(tbuf, [a0])
                    tv1 = plsc.load_gather(tbuf, [a1])
                    for b in range(B):
                        fs = fvecs[b][j]
                        d0 = fs - tv0
                        d1 = fs - tv1
                        if j == 0:
                            part[2 * b] = d0 * d0
                            part[2 * b + 1] = d1 * d1
                        else:
                            part[2 * b] = part[2 * b] + d0 * d0
                            part[2 * b + 1] = part[2 * b + 1] + d1 * d1
                accs = tuple(a + p for a, p in zip(accs, part))
                return idx + _c(8), accs

            zero = jnp.zeros((L,), F32)
            _, accs = lax.fori_loop(0, D // 8, k_step,
                                    (idx0, (zero,) * (2 * B)))
            for b in range(B):
                for h in range(2):
                    distbuf[pl.ds(b * (PXC * DSTR) + doff + h * L, L)] = \
                        accs[2 * b + h]
            return 0

        lax.fori_loop(0, PXC, px_step, 0)

    def phase2(op, h):
        for b0 in (0, 2):
            base = dpat + _c(b0 * (PXC * DSTR))
            bd = jnp.full((L,), jnp.inf, F32)
            bi = jnp.zeros((L,), I32)

            def t_step(q, carry):
                bd, bi = carry
                t = 4 * q
                v0 = plsc.load_gather(distbuf, [base + t])
                v1 = plsc.load_gather(distbuf, [base + (t + 1)])
                v2 = plsc.load_gather(distbuf, [base + (t + 2)])
                v3 = plsc.load_gather(distbuf, [base + (t + 3)])
                i01 = jnp.where(v1 < v0, t + 1, t)
                m01 = jnp.minimum(v0, v1)
                i23 = jnp.where(v3 < v2, t + 3, t + 2)
                m23 = jnp.minimum(v2, v3)
                lt2 = m23 < m01
                m4 = jnp.where(lt2, m23, m01)
                i4 = jnp.where(lt2, i23, i01)
                lt = m4 < bd
                return jnp.where(lt, m4, bd), jnp.where(lt, i4, bi)

            bd, bi = lax.fori_loop(0, T // 4, t_step, (bd, bi))
            mask = bd <= THRESH
            cls = plsc.load_gather(clsv, [bi])
            so = spat + _c(op * B * 2 * PXC + b0 * 2 * PXC + h * PXC)
            plsc.store_scatter(minb, [so], bd)
            plsc.store_scatter(maskb, [so], jnp.where(mask, 1, 0).astype(I32))
            plsc.store_scatter(nclsb, [so],
                               jnp.where(mask, cls, NCAT - 1).astype(I32))
            plsc.store_scatter(uclsb, [so], cls)
            po = ppat + _c(op * B * PSTR + b0 * PSTR + h * PXC * NCAT)
            for c in range(NCAT):
                pv = jnp.where((cls == c) & mask, 1.0, 0.0).astype(F32)
                plsc.store_scatter(predb, [po + _c(c)], pv)

    fire_frame(0, 0)
    fire_tpl(0, 0, 0)

    def outer(i, _):
        for s in range(8):
            tp = s % 2
            tgc = s % 2
            q = (s // 2) % 2
            op = s // 4
            if tgc == 0:
                wait_frame(q)
                if s // 2 == 3:
                    @pl.when(i < NPXC // 4 - 1)
                    def _():
                        fire_frame(4 * i + 4, 1 - q)
                else:
                    fire_frame(4 * i + s // 2 + 1, 1 - q)
            wait_tpl(tp)
            if s == 7:
                @pl.when(i < NPXC // 4 - 1)
                def _():
                    fire_tpl(4 * i + 4, 0, 0)
            else:
                fire_tpl(4 * i + (s + 1) // 2, (s + 1) % 2, 1 - tp)
            compute_chunk(tp, q, tgc)
            if tgc == 1:
                if s % 4 == 1:
                    @pl.when(i >= 1)
                    def _():
                        for cp in out_copies(op, 0):
                            cp.wait()
                phase2(op, (s // 2) % 2)
                if s % 4 == 3:
                    px0 = pxbase + (2 * i + s // 4) * 2 * PXC
                    for cp in out_copies(op, px0):
                        cp.start()
        return 0

    lax.fori_loop(0, NPXC // 4, outer, 0)

    for op in range(2):
        for cp in out_copies(op, 0):
            cp.wait()


@jax.jit
def _nn_classify(frame, tpl, clsa):
    mesh = plsc.VectorSubcoreMesh(core_axis_name="c", subcore_axis_name="s")
    fn = functools.partial(
        pl.kernel,
        out_type=(
            jax.ShapeDtypeStruct((FLAT * NCAT,), F32),
            jax.ShapeDtypeStruct((FLAT,), I32),
            jax.ShapeDtypeStruct((FLAT,), I32),
            jax.ShapeDtypeStruct((FLAT,), F32),
            jax.ShapeDtypeStruct((FLAT,), I32),
        ),
        mesh=mesh,
        compiler_params=pltpu.CompilerParams(needs_layout_passes=False),
        scratch_types=[
            pltpu.VMEM((2 * TB,), F32),
            pltpu.VMEM((2 * FB + 8,), F32),
            pltpu.VMEM((B * PXC * DSTR,), F32),
            pltpu.VMEM((T,), I32),
            pltpu.VMEM((2 * B * PSTR,), F32),
            pltpu.VMEM((2 * B * 2 * PXC,), F32),
            pltpu.VMEM((2 * B * 2 * PXC,), I32),
            pltpu.VMEM((2 * B * 2 * PXC,), I32),
            pltpu.VMEM((2 * B * 2 * PXC,), I32),
            pltpu.SemaphoreType.DMA((2,)),
            pltpu.SemaphoreType.DMA((2,)),
            pltpu.SemaphoreType.DMA((2,)),
        ],
    )(_body)
    return fn(frame, tpl, clsa)


TC_PX = HW - SC_PX
TC_TILE = 512
TC_NT = TC_PX // TC_TILE
TC_OFF = SC_PX // TC_TILE


def _tc_body(cls_ref, frame_ref, tpl_ref,
             pred_o, maski_o, ncls_o, mind_o, ucls_o, best, bestc):
    t = pl.program_id(1)
    tm = lax.broadcast_in_dim(tpl_ref[...], (B, TC_TILE, D), (0, 1, 2))
    diff = frame_ref[...] - tm
    d2 = jnp.sum(diff * diff, axis=2)
    ct = jnp.full((B, TC_TILE), cls_ref[t], I32)

    @pl.when(t == 0)
    def _():
        best[...] = d2
        bestc[...] = ct

    @pl.when(t > 0)
    def _():
        upd = d2 < best[...]
        best[...] = jnp.where(upd, d2, best[...])
        bestc[...] = jnp.where(upd, ct, bestc[...])

    @pl.when(t == T - 1)
    def _():
        bd = best[...]
        cls = bestc[...]
        m = bd <= THRESH
        mind_o[...] = bd
        maski_o[...] = m.astype(I32)
        ncls_o[...] = jnp.where(m, cls, NCAT - 1)
        ucls_o[...] = cls
        clsb = lax.broadcast_in_dim(cls, (B, NCAT, TC_TILE), (0, 2))
        mb = lax.broadcast_in_dim(m, (B, NCAT, TC_TILE), (0, 2))
        ioc = lax.broadcasted_iota(I32, (B, NCAT, TC_TILE), 1)
        pred_o[...] = jnp.where(jnp.logical_and(clsb == ioc, mb),
                                1.0, 0.0).astype(F32)


@jax.jit
def _tc_classify(frame, tpl, clsa):
    return pl.pallas_call(
        _tc_body,
        grid=(TC_NT, T),
        in_specs=[
            pl.BlockSpec(memory_space=pltpu.SMEM),
            pl.BlockSpec((B, TC_TILE, D), lambda pt, t: (0, TC_OFF + pt, 0)),
            pl.BlockSpec((1, TC_TILE, D), lambda pt, t: (t, TC_OFF + pt, 0)),
        ],
        out_specs=[
            pl.BlockSpec((B, NCAT, TC_TILE), lambda pt, t: (0, 0, pt)),
            pl.BlockSpec((B, TC_TILE), lambda pt, t: (0, pt)),
            pl.BlockSpec((B, TC_TILE), lambda pt, t: (0, pt)),
            pl.BlockSpec((B, TC_TILE), lambda pt, t: (0, pt)),
            pl.BlockSpec((B, TC_TILE), lambda pt, t: (0, pt)),
        ],
        out_shape=(
            jax.ShapeDtypeStruct((B, NCAT, TC_PX), F32),
            jax.ShapeDtypeStruct((B, TC_PX), I32),
            jax.ShapeDtypeStruct((B, TC_PX), I32),
            jax.ShapeDtypeStruct((B, TC_PX), F32),
            jax.ShapeDtypeStruct((B, TC_PX), I32),
        ),
        scratch_shapes=[
            pltpu.VMEM((B, TC_TILE), F32),
            pltpu.VMEM((B, TC_TILE), I32),
        ],
    )(clsa, frame, tpl)


def kernel(frame_embeddings, templates, template_classes):
    pred, maski, ncls, mind, ucls = _nn_classify(
        frame_embeddings.reshape(B * HW * D),
        templates.reshape(T * HW * D),
        template_classes)
    tpred, tmaski, tncls, tmind, tucls = _tc_classify(
        frame_embeddings, templates, template_classes)
    cat = lambda a, b: jnp.concatenate([a, b], axis=1)
    return (
        cat(pred.reshape(B, HW, NCAT)[:, :SC_PX],
            jnp.transpose(tpred, (0, 2, 1))),
        cat(maski.reshape(B, HW)[:, :SC_PX], tmaski).astype(bool),
        cat(ncls.reshape(B, HW)[:, :SC_PX], tncls),
        cat(mind.reshape(B, HW)[:, :SC_PX], tmind),
        cat(ucls.reshape(B, HW)[:, :SC_PX], tucls),
    )
